# baseline (speedup 1.0000x reference)
import functools

import jax
import jax.numpy as jnp
from jax import lax
from jax.experimental import pallas as pl
from jax.experimental.pallas import tpu as pltpu
from jax.experimental.pallas import tpu_sc as plsc

D = 768
LANES = 16
NG = D // LANES
NC = 2
NS = 16
NW = NC * NS


@functools.lru_cache(maxsize=None)
def _make_sc_kernel(n_rows, seq):
    chunk = n_rows // NW
    sub = 32
    nsub = chunk // sub
    mesh = plsc.VectorSubcoreMesh(core_axis_name="c", subcore_axis_name="s",
                                  num_cores=NC, num_subcores=NS)

    @functools.partial(
        pl.kernel,
        out_type=jax.ShapeDtypeStruct((n_rows, D), jnp.float32),
        mesh=mesh,
        scratch_types=[
            pltpu.VMEM((sub,), jnp.int32),
            pltpu.VMEM((sub, D), jnp.float32),
            pltpu.VMEM((sub, D), jnp.float32),
            pltpu.VMEM((5, D), jnp.float32),
            pltpu.VMEM((D,), jnp.float32),
            pltpu.SemaphoreType.DMA,
        ],
    )
    def sc_kernel(ids_hbm, table_hbm, pos_hbm, dim_hbm, out_hbm,
                  idx_v, rows_v, pos_v, dim_v, cvec_v, sem):
        wid = lax.axis_index("s") * NC + lax.axis_index("c")
        base = wid * chunk
        s_base = base % seq

        pltpu.sync_copy(dim_hbm, dim_v)
        for g in range(NG):
            sl = pl.ds(g * LANES, LANES)
            acc = (dim_v[0, sl] + dim_v[1, sl] + dim_v[2, sl]
                   + dim_v[3, sl] + dim_v[4, sl])
            cvec_v[sl] = acc * 0.2

        for j in range(nsub):
            off = j * sub
            pltpu.sync_copy(ids_hbm.at[pl.ds(base + off, sub)], idx_v)
            pltpu.async_copy(table_hbm.at[idx_v], rows_v, sem).wait()
            pltpu.sync_copy(pos_hbm.at[pl.ds(s_base + off, sub)], pos_v)

            def row_add(r, _):
                for g in range(NG):
                    sl = pl.ds(g * LANES, LANES)
                    rows_v[r, sl] = rows_v[r, sl] + pos_v[r, sl] + cvec_v[sl]
                return 0

            lax.fori_loop(0, sub, row_add, 0)
            pltpu.sync_copy(rows_v, out_hbm.at[pl.ds(base + off, sub)])

    return sc_kernel


def kernel(input_ids, table, pos_enc, dim_emb):
    b, s = input_ids.shape
    ids = input_ids.reshape(-1).astype(jnp.int32)
    sc = _make_sc_kernel(b * s, s)
    out = sc(ids, table, pos_enc, dim_emb)
    return out.reshape(b, s, D)


# speedup vs baseline: 1.2369x; 1.2369x over previous
import functools

import jax
import jax.numpy as jnp
from jax import lax
from jax.experimental import pallas as pl
from jax.experimental.pallas import tpu as pltpu
from jax.experimental.pallas import tpu_sc as plsc

D = 768
LANES = 16
NG = D // LANES
NC = 2
NS = 16
NW = NC * NS


@functools.lru_cache(maxsize=None)
def _make_sc_kernel(n_rows, seq):
    chunk = n_rows // NW
    sub = 32
    nsub = chunk // sub
    mesh = plsc.VectorSubcoreMesh(core_axis_name="c", subcore_axis_name="s",
                                  num_cores=NC, num_subcores=NS)

    @functools.partial(
        pl.kernel,
        out_type=jax.ShapeDtypeStruct((n_rows, D), jnp.float32),
        mesh=mesh,
        scratch_types=[
            pltpu.VMEM((nsub, sub), jnp.int32),
            pltpu.VMEM((sub, D), jnp.float32),
            pltpu.VMEM((sub, D), jnp.float32),
            pltpu.VMEM((sub, D), jnp.float32),
            pltpu.VMEM((sub, D), jnp.float32),
            pltpu.VMEM((5, D), jnp.float32),
            pltpu.VMEM((D,), jnp.float32),
            pltpu.SemaphoreType.DMA,
            pltpu.SemaphoreType.DMA,
            pltpu.SemaphoreType.DMA,
            pltpu.SemaphoreType.DMA,
            pltpu.SemaphoreType.DMA,
            pltpu.SemaphoreType.DMA,
        ],
    )
    def sc_kernel(ids_hbm, table_hbm, pos_hbm, dim_hbm, out_hbm,
                  idx_all, rows0, rows1, pos0, pos1, dim_v, cvec_v,
                  gsem0, gsem1, psem0, psem1, osem0, osem1):
        rows = (rows0, rows1)
        poss = (pos0, pos1)
        gsems = (gsem0, gsem1)
        psems = (psem0, psem1)
        osems = (osem0, osem1)

        wid = lax.axis_index("s") * NC + lax.axis_index("c")
        base = wid * chunk
        s_base = base % seq

        pltpu.sync_copy(ids_hbm.at[pl.ds(wid * nsub, nsub)], idx_all)

        pltpu.sync_copy(dim_hbm, dim_v)
        for g in range(NG):
            sl = pl.ds(g * LANES, LANES)
            acc = (dim_v[0, sl] + dim_v[1, sl] + dim_v[2, sl]
                   + dim_v[3, sl] + dim_v[4, sl])
            cvec_v[sl] = acc * 0.2

        def start_fetch(j):
            b = j % 2
            gh = pltpu.async_copy(table_hbm.at[idx_all.at[j]], rows[b],
                                  gsems[b])
            ph = pltpu.async_copy(pos_hbm.at[pl.ds(s_base + j * sub, sub)],
                                  poss[b], psems[b])
            return gh, ph

        handles = [None] * nsub
        out_handles = [None] * nsub
        handles[0] = start_fetch(0)

        for j in range(nsub):
            b = j % 2
            nb = 1 - b
            if j + 1 < nsub:
                if j >= 1:
                    out_handles[j - 1].wait()
                handles[j + 1] = start_fetch(j + 1)
            gh, ph = handles[j]
            gh.wait()
            ph.wait()

            def row_add(r, _, rv=rows[b], pv=poss[b]):
                for g in range(NG):
                    sl = pl.ds(g * LANES, LANES)
                    rv[r, sl] = rv[r, sl] + pv[r, sl] + cvec_v[sl]
                return 0

            lax.fori_loop(0, sub, row_add, 0)
            out_handles[j] = pltpu.async_copy(
                rows[b], out_hbm.at[pl.ds(base + j * sub, sub)], osems[b])

        out_handles[nsub - 2].wait()
        out_handles[nsub - 1].wait()

    return sc_kernel


def kernel(input_ids, table, pos_enc, dim_emb):
    b, s = input_ids.shape
    ids = input_ids.reshape(-1, 32).astype(jnp.int32)
    sc = _make_sc_kernel(b * s, s)
    out = sc(ids, table, pos_enc, dim_emb)
    return out.reshape(b, s, D)
